# Initial kernel scaffold; baseline (speedup 1.0000x reference)
import jax
import jax.numpy as jnp
from jax.experimental import pallas as pl


def kernel(species, r_ij_len, idx_i, idx_j, embeddings_flat):
    raise NotImplementedError("write your pallas kernel here")


# speedup vs baseline: 21.1977x; 21.1977x over previous
import functools

import jax
import jax.numpy as jnp
import numpy as np
from jax import lax
from jax.experimental import pallas as pl
from jax.experimental.pallas import tpu as pltpu
from jax.experimental.pallas import tpu_sc as plsc

N_SPECIES = 119
N_RADIAL = 5
N_BASIS = 7
R_CUTOFF = 5.0
R_MIN = 0.5
N_NODES = 100000
N_EDGES = 1600000

_BETTA = float(N_BASIS) ** 2 / R_CUTOFF ** 2
_NORM = (2.0 * _BETTA / np.pi) ** 0.25
_SCALE_EMB = 1.0 / float(np.sqrt(N_BASIS))
_OUT_SCALE = float(_NORM * _SCALE_EMB)
_SHIFTS = [R_MIN + i * (R_CUTOFF - R_MIN) / N_BASIS for i in range(N_BASIS)]

_NC, _NS, _L = 2, 16, 16
_NW = _NC * _NS
_D = N_RADIAL * N_BASIS
_DP = 40


def _build(n_nodes, n_edges, e_chunk, sub, interpret=False):
    n_sub = e_chunk // sub
    n_groups = e_chunk // _L
    chunks_per_w = n_edges // (_NW * e_chunk)
    assert n_edges == _NW * e_chunk * chunks_per_w
    assert sub % _L == 0 and e_chunk % sub == 0

    def body(species_hbm, r_hbm, ii_hbm, jj_hbm, emb_hbm, out_hbm,
             species_v, ii_v, jj_v, r_v, pair_v, rows_v, out_v, sem):
        wid = lax.axis_index("s") * _NC + lax.axis_index("c")
        pltpu.sync_copy(species_hbm, species_v)
        iota16 = lax.iota(jnp.int32, _L)

        def chunk(c, carry):
            base = (wid * chunks_per_w + c) * e_chunk
            pltpu.sync_copy(ii_hbm.at[pl.ds(base, e_chunk)], ii_v)
            pltpu.sync_copy(jj_hbm.at[pl.ds(base, e_chunk)], jj_v)
            pltpu.sync_copy(r_hbm.at[pl.ds(base, e_chunk)], r_v)

            gpr = sub // _L
            for g in range(n_groups):
                ii = ii_v[pl.ds(g * _L, _L)]
                jj = jj_v[pl.ds(g * _L, _L)]
                si = plsc.load_gather(species_v, [ii])
                sj = plsc.load_gather(species_v, [jj])
                pair = si * N_SPECIES + sj
                pair_v[g // gpr, pl.ds((g % gpr) * _L, _L)] = pair

            cps = [
                pltpu.async_copy(emb_hbm.at[pair_v.at[j]],
                                 rows_v.at[pl.ds(j * sub, sub)], sem)
                for j in range(n_sub)
            ]
            for cp in cps:
                cp.wait()

            for g in range(n_groups):
                e_ids = g * _L + iota16
                r = r_v[pl.ds(g * _L, _L)]
                es = []
                for b in range(N_BASIS):
                    d = r - _SHIFTS[b]
                    es.append(jnp.exp(d * d * (-_BETTA)))
                for rr in range(N_RADIAL):
                    acc = None
                    for b in range(N_BASIS):
                        col = jnp.full((_L,), rr * N_BASIS + b, jnp.int32)
                        cv = plsc.load_gather(rows_v, [e_ids, col])
                        t = cv * es[b]
                        acc = t if acc is None else acc + t
                    outv = acc * _OUT_SCALE
                    plsc.store_scatter(
                        out_v, [e_ids, jnp.full((_L,), rr, jnp.int32)], outv)

            pltpu.sync_copy(out_v, out_hbm.at[pl.ds(base, e_chunk)])
            return carry

        lax.fori_loop(0, chunks_per_w, chunk, 0)

    return functools.partial(
        pl.kernel,
        mesh=plsc.VectorSubcoreMesh(core_axis_name="c", subcore_axis_name="s"),
        out_type=jax.ShapeDtypeStruct((n_edges, N_RADIAL), jnp.float32),
        compiler_params=pltpu.CompilerParams(needs_layout_passes=False,
                                             use_tc_tiling_on_sc=False),
        interpret=interpret,
        scratch_types=[
            pltpu.VMEM((n_nodes,), jnp.int32),
            pltpu.VMEM((e_chunk,), jnp.int32),
            pltpu.VMEM((e_chunk,), jnp.int32),
            pltpu.VMEM((e_chunk,), jnp.float32),
            pltpu.VMEM((n_sub, sub), jnp.int32),
            pltpu.VMEM((e_chunk, _DP), jnp.float32),
            pltpu.VMEM((e_chunk, N_RADIAL), jnp.float32),
            pltpu.SemaphoreType.DMA,
        ],
    )(body)


_rbf = _build(N_NODES, N_EDGES, 400, 80)


@jax.jit
def kernel(species, r_ij_len, idx_i, idx_j, embeddings_flat):
    emb2d = jnp.pad(embeddings_flat.reshape(N_SPECIES * N_SPECIES, _D),
                    ((0, 0), (0, _DP - _D)))
    return _rbf(species.astype(jnp.int32), r_ij_len,
                idx_i.astype(jnp.int32), idx_j.astype(jnp.int32), emb2d)


# speedup vs baseline: 32.0926x; 1.5140x over previous
import functools

import jax
import jax.numpy as jnp
import numpy as np
from jax import lax
from jax.experimental import pallas as pl
from jax.experimental.pallas import tpu as pltpu
from jax.experimental.pallas import tpu_sc as plsc

N_SPECIES = 119
N_RADIAL = 5
N_BASIS = 7
R_CUTOFF = 5.0
R_MIN = 0.5
N_NODES = 100000
N_EDGES = 1600000

_BETTA = float(N_BASIS) ** 2 / R_CUTOFF ** 2
_NORM = (2.0 * _BETTA / np.pi) ** 0.25
_SCALE_EMB = 1.0 / float(np.sqrt(N_BASIS))
_OUT_SCALE = float(_NORM * _SCALE_EMB)
_SHIFTS = [R_MIN + i * (R_CUTOFF - R_MIN) / N_BASIS for i in range(N_BASIS)]

_NC, _NS, _L = 2, 16, 16
_NW = _NC * _NS
_D = N_RADIAL * N_BASIS
_DP = 40
_NPK = N_NODES // 4


def _build(n_nodes, n_edges, e_chunk, sub, interpret=False):
    n_sub = e_chunk // sub
    n_groups = e_chunk // _L
    gps = sub // _L
    chunks_per_w = n_edges // (_NW * e_chunk)
    assert n_edges == _NW * e_chunk * chunks_per_w
    assert sub % _L == 0 and e_chunk % sub == 0 and n_nodes % 4 == 0
    npk = n_nodes // 4

    def body(spk_hbm, r_hbm, ii_hbm, jj_hbm, emb_hbm, out_hbm,
             spk_v, ii_v, jj_v, r_v, pair_v, rows_v, out_v, sem_in, sem_g):
        wid = lax.axis_index("s") * _NC + lax.axis_index("c")
        pltpu.sync_copy(spk_hbm, spk_v)
        iota16 = lax.iota(jnp.int32, _L)

        def lookup(idx):
            w = plsc.load_gather(spk_v, [idx >> 2])
            sh = (idx & 3) << 3
            return (w >> sh) & 0xFF

        def chunk(c, carry):
            base = (wid * chunks_per_w + c) * e_chunk
            cp1 = pltpu.async_copy(ii_hbm.at[pl.ds(base, e_chunk)], ii_v, sem_in)
            cp2 = pltpu.async_copy(jj_hbm.at[pl.ds(base, e_chunk)], jj_v, sem_in)
            cp3 = pltpu.async_copy(r_hbm.at[pl.ds(base, e_chunk)], r_v, sem_in)
            cp1.wait(); cp2.wait(); cp3.wait()

            def pairs(g, cy):
                ii = ii_v[pl.ds(g * _L, _L)]
                jj = jj_v[pl.ds(g * _L, _L)]
                pair = lookup(ii) * N_SPECIES + lookup(jj)
                pair_v[g // gps, pl.ds((g % gps) * _L, _L)] = pair
                return cy

            lax.fori_loop(0, n_groups, pairs, 0)

            def fire(j, cy):
                pltpu.async_copy(emb_hbm.at[pair_v.at[j]],
                                 rows_v.at[pl.ds(j * sub, sub)], sem_g)
                return cy

            lax.fori_loop(0, n_sub, fire, 0)

            def drain(j, cy):
                pltpu.make_async_copy(emb_hbm.at[pl.ds(0, sub)],
                                      rows_v.at[pl.ds(0, sub)], sem_g).wait()
                for u in range(gps):
                    g = j * gps + u
                    e_ids = g * _L + iota16
                    r = r_v[pl.ds(g * _L, _L)]
                    es = []
                    for b in range(N_BASIS):
                        d = r - _SHIFTS[b]
                        es.append(jnp.exp(d * d * (-_BETTA)))
                    for rr in range(N_RADIAL):
                        acc = None
                        for b in range(N_BASIS):
                            col = jnp.full((_L,), rr * N_BASIS + b, jnp.int32)
                            cv = plsc.load_gather(rows_v, [e_ids, col])
                            t = cv * es[b]
                            acc = t if acc is None else acc + t
                        plsc.store_scatter(
                            out_v, [e_ids, jnp.full((_L,), rr, jnp.int32)],
                            acc * _OUT_SCALE)
                return cy

            lax.fori_loop(0, n_sub, drain, 0)

            pltpu.sync_copy(out_v, out_hbm.at[pl.ds(base, e_chunk)])
            return carry

        lax.fori_loop(0, chunks_per_w, chunk, 0)

    return functools.partial(
        pl.kernel,
        mesh=plsc.VectorSubcoreMesh(core_axis_name="c", subcore_axis_name="s"),
        out_type=jax.ShapeDtypeStruct((n_edges, N_RADIAL), jnp.float32),
        compiler_params=pltpu.CompilerParams(needs_layout_passes=False,
                                             use_tc_tiling_on_sc=False),
        interpret=interpret,
        scratch_types=[
            pltpu.VMEM((npk,), jnp.int32),
            pltpu.VMEM((e_chunk,), jnp.int32),
            pltpu.VMEM((e_chunk,), jnp.int32),
            pltpu.VMEM((e_chunk,), jnp.float32),
            pltpu.VMEM((n_sub, sub), jnp.int32),
            pltpu.VMEM((e_chunk, _DP), jnp.float32),
            pltpu.VMEM((e_chunk, N_RADIAL), jnp.float32),
            pltpu.SemaphoreType.DMA,
            pltpu.SemaphoreType.DMA,
        ],
    )(body)


_rbf = _build(N_NODES, N_EDGES, 2000, 80)


@jax.jit
def kernel(species, r_ij_len, idx_i, idx_j, embeddings_flat):
    emb2d = jnp.pad(embeddings_flat.reshape(N_SPECIES * N_SPECIES, _D),
                    ((0, 0), (0, _DP - _D)))
    sp = species.astype(jnp.int32).reshape(_NPK, 4)
    spk = (sp[:, 0] | (sp[:, 1] << 8) | (sp[:, 2] << 16) | (sp[:, 3] << 24))
    return _rbf(spk, r_ij_len,
                idx_i.astype(jnp.int32), idx_j.astype(jnp.int32), emb2d)


# speedup vs baseline: 92.5616x; 2.8842x over previous
import functools

import jax
import jax.numpy as jnp
import numpy as np
from jax import lax
from jax.experimental import pallas as pl
from jax.experimental.pallas import tpu as pltpu
from jax.experimental.pallas import tpu_sc as plsc

N_SPECIES = 119
N_RADIAL = 5
N_BASIS = 7
R_CUTOFF = 5.0
R_MIN = 0.5
N_NODES = 100000
N_EDGES = 1600000

_BETTA = float(N_BASIS) ** 2 / R_CUTOFF ** 2
_NORM = (2.0 * _BETTA / np.pi) ** 0.25
_SCALE_EMB = 1.0 / float(np.sqrt(N_BASIS))
_LN_S = float(np.log(_NORM * _SCALE_EMB))
_SHIFTS = [R_MIN + i * (R_CUTOFF - R_MIN) / N_BASIS for i in range(N_BASIS)]

_NC, _NS, _L = 2, 16, 16
_NW = _NC * _NS
_BP = 8
_DW = 24
_NPK = N_NODES // 4
_TILE = 128
_PR = 8


def _build(n_nodes, n_edges, e_chunk, sub, interpret=False):
    n_sub = e_chunk // sub
    gps = sub // _L
    tiles_per_chunk = e_chunk // _TILE
    n_chunks = n_edges // e_chunk
    n_tiles = n_edges // _TILE
    assert n_edges == n_chunks * e_chunk and e_chunk % _TILE == 0
    assert sub % _L == 0 and e_chunk % sub == 0 and n_nodes % 4 == 0
    npk = n_nodes // 4
    base_chunks = n_chunks // _NW
    extra = n_chunks % _NW

    def body(spk_hbm, r_hbm, ii_hbm, jj_hbm, emb_hbm, out_hbm,
             spk_v, in_v, r_v, pair_v, rows_v, out_v, sem_in, sem_g, sem_o):
        wid = lax.axis_index("s") * _NC + lax.axis_index("c")
        pltpu.sync_copy(spk_hbm, spk_v)
        iota16 = lax.iota(jnp.int32, _L)
        zeros16 = jnp.zeros((_L,), jnp.float32)
        my_chunks = base_chunks + jnp.where(wid < extra, 1, 0)

        for p in range(2):
            for t in range(tiles_per_chunk):
                for rr in range(N_RADIAL, _PR):
                    for blk in range(_TILE // _L):
                        out_v[p, t, rr, pl.ds(blk * _L, _L)] = zeros16

        def lookup(idx):
            w = plsc.load_gather(spk_v, [idx >> 2])
            sh = (idx & 3) << 3
            return (w >> sh) & 0xFF

        def fetch(c, p):
            base = (c * _NW + wid) * e_chunk
            pltpu.async_copy(ii_hbm.at[pl.ds(base, e_chunk)],
                             in_v.at[p, 0], sem_in)
            pltpu.async_copy(jj_hbm.at[pl.ds(base, e_chunk)],
                             in_v.at[p, 1], sem_in)
            pltpu.async_copy(r_hbm.at[pl.ds(base, e_chunk)],
                             r_v.at[p], sem_in)

        fetch(0, 0)

        def chunk(c, carry):
            p = c % 2
            cid = c * _NW + wid
            pltpu.make_async_copy(ii_hbm.at[pl.ds(0, e_chunk)],
                                  in_v.at[0, 0], sem_in).wait()
            pltpu.make_async_copy(ii_hbm.at[pl.ds(0, e_chunk)],
                                  in_v.at[0, 1], sem_in).wait()
            pltpu.make_async_copy(r_hbm.at[pl.ds(0, e_chunk)],
                                  r_v.at[0], sem_in).wait()

            @pl.when(c + 1 < my_chunks)
            def _():
                fetch(c + 1, 1 - p)

            @pl.when(c >= 2)
            def _():
                pltpu.make_async_copy(
                    out_v.at[0], out_hbm.at[pl.ds(0, tiles_per_chunk)],
                    sem_o).wait()

            def prep(j, cy):
                for u in range(gps):
                    g = j * gps + u
                    ii = in_v[p, 0, pl.ds(g * _L, _L)]
                    jj = in_v[p, 1, pl.ds(g * _L, _L)]
                    pair = lookup(ii) * N_SPECIES + lookup(jj)
                    pair_v[j, pl.ds(u * _L, _L)] = pair
                pltpu.async_copy(emb_hbm.at[pair_v.at[j]],
                                 rows_v.at[pl.ds(j * sub, sub)], sem_g)
                return cy

            lax.fori_loop(0, n_sub, prep, 0)

            def drain(j, cy):
                pltpu.make_async_copy(emb_hbm.at[pl.ds(0, sub)],
                                      rows_v.at[pl.ds(0, sub)], sem_g).wait()
                for u in range(gps):
                    g = j * gps + u
                    e_ids = g * _L + iota16
                    r = r_v[p, pl.ds(g * _L, _L)]
                    es = []
                    for b in range(N_BASIS):
                        d = r - _SHIFTS[b]
                        es.append(jnp.exp(d * d * (-_BETTA) + _LN_S))
                    t = g // (_TILE // _L)
                    cb = (g % (_TILE // _L)) * _L
                    for rr in range(N_RADIAL):
                        acc = None
                        for k in range(4):
                            w = plsc.load_gather(
                                rows_v,
                                [e_ids, jnp.full((_L,), rr * 4 + k,
                                                 jnp.int32)])
                            lo = plsc.bitcast(w << 16, jnp.float32)
                            tv = lo * es[2 * k]
                            acc = tv if acc is None else acc + tv
                            if 2 * k + 1 < N_BASIS:
                                hi = plsc.bitcast(w & jnp.int32(-65536),
                                                  jnp.float32)
                                acc = acc + hi * es[2 * k + 1]
                        out_v[p, t, rr, pl.ds(cb, _L)] = acc
                return cy

            lax.fori_loop(0, n_sub, drain, 0)

            pltpu.async_copy(out_v.at[p],
                             out_hbm.at[pl.ds(cid * tiles_per_chunk,
                                              tiles_per_chunk)], sem_o)
            return carry

        lax.fori_loop(0, my_chunks, chunk, 0)

        for _ in range(2):
            pltpu.make_async_copy(out_v.at[0],
                                  out_hbm.at[pl.ds(0, tiles_per_chunk)],
                                  sem_o).wait()

    return functools.partial(
        pl.kernel,
        mesh=plsc.VectorSubcoreMesh(core_axis_name="c", subcore_axis_name="s"),
        out_type=jax.ShapeDtypeStruct((n_tiles, _PR, _TILE), jnp.float32),
        compiler_params=pltpu.CompilerParams(needs_layout_passes=False,
                                             use_tc_tiling_on_sc=False),
        interpret=interpret,
        scratch_types=[
            pltpu.VMEM((npk,), jnp.int32),
            pltpu.VMEM((2, 2, e_chunk), jnp.int32),
            pltpu.VMEM((2, e_chunk), jnp.float32),
            pltpu.VMEM((n_sub, sub), jnp.int32),
            pltpu.VMEM((e_chunk, _DW), jnp.int32),
            pltpu.VMEM((2, tiles_per_chunk, _PR, _TILE), jnp.float32),
            pltpu.SemaphoreType.DMA,
            pltpu.SemaphoreType.DMA,
            pltpu.SemaphoreType.DMA,
        ],
    )(body)


_rbf = _build(N_NODES, N_EDGES, 1280, 128)


@jax.jit
def kernel(species, r_ij_len, idx_i, idx_j, embeddings_flat):
    embh = jnp.pad(embeddings_flat.astype(jnp.bfloat16),
                   ((0, 0), (0, 0), (0, _BP - N_BASIS)))
    embw = lax.bitcast_convert_type(
        embh.reshape(-1, N_RADIAL, _BP // 2, 2), jnp.int32)
    emb3 = jnp.pad(embw.reshape(-1, N_RADIAL * _BP // 2),
                   ((0, 0), (0, _DW - N_RADIAL * _BP // 2)))
    sp = species.astype(jnp.int32)
    spk = (sp[0::4] | (sp[1::4] << 8) | (sp[2::4] << 16) | (sp[3::4] << 24))
    out3 = _rbf(spk, r_ij_len,
                idx_i.astype(jnp.int32), idx_j.astype(jnp.int32), emb3)
    return out3.transpose(1, 0, 2).reshape(_PR, N_EDGES).T[:, :N_RADIAL]
